# baseline (speedup 1.0000x reference)
import functools

import jax
import jax.numpy as jnp
from jax.experimental import pallas as pl

B, C, H, W_SPATIAL = 1024, 768, 14, 14
HW = H * W_SPATIAL
NUM_CLUSTER = 10
BLOCK_B = 8


def _sem_head_kernel(fea_ref, w_ref, b_ref, out_ref):
    x = fea_ref[...]
    feat = jnp.sum(x, axis=2) * (1.0 / HW)
    score = jax.lax.dot_general(
        feat, w_ref[...],
        dimension_numbers=(((1,), (1,)), ((), ())),
        preferred_element_type=jnp.float32,
    )
    out_ref[...] = score + b_ref[...]


@jax.jit
def kernel(fea, W, b):
    fea3 = fea.reshape(B, C, HW)
    b2 = b.reshape(1, NUM_CLUSTER)
    grid = (B // BLOCK_B,)
    return pl.pallas_call(
        _sem_head_kernel,
        grid=grid,
        in_specs=[
            pl.BlockSpec((BLOCK_B, C, HW), lambda i: (i, 0, 0)),
            pl.BlockSpec((NUM_CLUSTER, C), lambda i: (0, 0)),
            pl.BlockSpec((1, NUM_CLUSTER), lambda i: (0, 0)),
        ],
        out_specs=pl.BlockSpec((BLOCK_B, NUM_CLUSTER), lambda i: (i, 0)),
        out_shape=jax.ShapeDtypeStruct((B, NUM_CLUSTER), jnp.float32),
    )(fea3, W, b2)
